# Initial kernel scaffold; baseline (speedup 1.0000x reference)
import jax
import jax.numpy as jnp
from jax.experimental import pallas as pl


def kernel(input, table):
    raise NotImplementedError("write your pallas kernel here")


# speedup vs baseline: 2.2833x; 2.2833x over previous
import functools

import jax
import jax.numpy as jnp
from jax import lax
from jax.experimental import pallas as pl
from jax.experimental.pallas import tpu as pltpu
from jax.experimental.pallas import tpu_sc as plsc

_PAD = 1
_NC = 2
_NS = 16
_NW = _NC * _NS
_LANE = 128
_CHUNK = 128


def _positions_body(x_ref, o_ref):
    x = x_ref[...]
    b, s = x.shape
    nchunk = s // _LANE
    mask = (x != _PAD).astype(jnp.float32)
    m = mask.reshape(b * nchunk, _LANE)
    r = lax.broadcasted_iota(jnp.int32, (_LANE, _LANE), 0)
    c = lax.broadcasted_iota(jnp.int32, (_LANE, _LANE), 1)
    incl = (r <= c).astype(jnp.float32)
    y = jnp.dot(m, incl, preferred_element_type=jnp.float32)
    sums = jnp.sum(mask.reshape(b, nchunk, _LANE), axis=-1)
    r2 = lax.broadcasted_iota(jnp.int32, (nchunk, nchunk), 0)
    c2 = lax.broadcasted_iota(jnp.int32, (nchunk, nchunk), 1)
    excl = (r2 < c2).astype(jnp.float32)
    off = jnp.dot(sums, excl, preferred_element_type=jnp.float32)
    pos = y.reshape(b, nchunk, _LANE) + off[:, :, None]
    pos = pos.reshape(b, s) * mask + float(_PAD)
    o_ref[...] = pos.astype(jnp.int32)


def _gather_rows(table, idx):
    v, d = table.shape
    n = idx.shape[0]
    b_per_w = n // _NW
    mesh = plsc.VectorSubcoreMesh(core_axis_name="c", subcore_axis_name="s")

    @functools.partial(
        pl.kernel,
        mesh=mesh,
        out_type=jax.ShapeDtypeStruct((n, d), jnp.float32),
        scratch_types=[
            pltpu.VMEM((b_per_w,), jnp.int32),
            pltpu.VMEM((_CHUNK, d), jnp.float32),
            pltpu.SemaphoreType.DMA,
        ],
    )
    def k(table_hbm, idx_hbm, out_hbm, idx_v, rows_v, sem):
        wid = lax.axis_index("s") * _NC + lax.axis_index("c")
        base = wid * b_per_w
        pltpu.sync_copy(idx_hbm.at[pl.ds(base, b_per_w)], idx_v)

        @pl.loop(0, b_per_w, step=_CHUNK)
        def _(c0):
            pltpu.async_copy(
                table_hbm.at[idx_v.at[pl.ds(c0, _CHUNK)]], rows_v, sem
            ).wait()
            pltpu.sync_copy(rows_v, out_hbm.at[pl.ds(base + c0, _CHUNK)])

    return k(table, idx)


def kernel(input, table):
    b, s = input.shape
    positions = pl.pallas_call(
        _positions_body,
        out_shape=jax.ShapeDtypeStruct((b, s), jnp.int32),
    )(input)
    idx = positions.reshape(-1)
    out = _gather_rows(table, idx)
    return out.reshape(b, s, table.shape[1])


# speedup vs baseline: 2.3922x; 1.0477x over previous
import functools

import jax
import jax.numpy as jnp
from jax import lax
from jax.experimental import pallas as pl
from jax.experimental.pallas import tpu as pltpu
from jax.experimental.pallas import tpu_sc as plsc

_PAD = 1
_NC = 2
_NS = 16
_NW = _NC * _NS
_LANE = 128
_CHUNK = 64


def _positions_body(x_ref, o_ref):
    x = x_ref[...]
    b, s = x.shape
    nchunk = s // _LANE
    mask = (x != _PAD).astype(jnp.float32)
    m = mask.reshape(b * nchunk, _LANE)
    r = lax.broadcasted_iota(jnp.int32, (_LANE, _LANE), 0)
    c = lax.broadcasted_iota(jnp.int32, (_LANE, _LANE), 1)
    incl = (r <= c).astype(jnp.float32)
    y = jnp.dot(m, incl, preferred_element_type=jnp.float32)
    sums = jnp.sum(mask.reshape(b, nchunk, _LANE), axis=-1)
    r2 = lax.broadcasted_iota(jnp.int32, (nchunk, nchunk), 0)
    c2 = lax.broadcasted_iota(jnp.int32, (nchunk, nchunk), 1)
    excl = (r2 < c2).astype(jnp.float32)
    off = jnp.dot(sums, excl, preferred_element_type=jnp.float32)
    pos = y.reshape(b, nchunk, _LANE) + off[:, :, None]
    pos = pos.reshape(b, s) * mask + float(_PAD)
    o_ref[...] = pos.astype(jnp.int32)


def _gather_rows(table, idx):
    v, d = table.shape
    n = idx.shape[0]
    b_per_w = n // _NW
    mesh = plsc.VectorSubcoreMesh(core_axis_name="c", subcore_axis_name="s")

    @functools.partial(
        pl.kernel,
        mesh=mesh,
        out_type=jax.ShapeDtypeStruct((n, d), jnp.float32),
        scratch_types=[
            pltpu.VMEM((b_per_w,), jnp.int32),
            pltpu.VMEM((_CHUNK, d), jnp.float32),
            pltpu.VMEM((_CHUNK, d), jnp.float32),
            pltpu.SemaphoreType.DMA,
            pltpu.SemaphoreType.DMA,
        ],
    )
    def k(table_hbm, idx_hbm, out_hbm, idx_v, rows0, rows1, sem0, sem1):
        wid = lax.axis_index("s") * _NC + lax.axis_index("c")
        base = wid * b_per_w
        pltpu.sync_copy(idx_hbm.at[pl.ds(base, b_per_w)], idx_v)

        rows = (rows0, rows1)
        sems = (sem0, sem1)
        nchunks = b_per_w // _CHUNK

        def gather(i):
            return pltpu.async_copy(
                table_hbm.at[idx_v.at[pl.ds(i * _CHUNK, _CHUNK)]],
                rows[i % 2],
                sems[i % 2],
            )

        handle = gather(0)
        for i in range(nchunks):
            nxt = gather(i + 1) if i + 1 < nchunks else None
            handle.wait()
            pltpu.sync_copy(
                rows[i % 2], out_hbm.at[pl.ds(base + i * _CHUNK, _CHUNK)]
            )
            handle = nxt

    return k(table, idx)


def kernel(input, table):
    b, s = input.shape
    positions = pl.pallas_call(
        _positions_body,
        out_shape=jax.ShapeDtypeStruct((b, s), jnp.int32),
    )(input)
    idx = positions.reshape(-1)
    out = _gather_rows(table, idx)
    return out.reshape(b, s, table.shape[1])


# speedup vs baseline: 2.3995x; 1.0030x over previous
import functools

import jax
import jax.numpy as jnp
from jax import lax
from jax.experimental import pallas as pl
from jax.experimental.pallas import tpu as pltpu
from jax.experimental.pallas import tpu_sc as plsc

_PAD = 1
_NC = 2
_NS = 16
_NW = _NC * _NS
_LANE = 128
_CHUNK = 64


def _positions_body(x_ref, o_ref):
    x = x_ref[...]
    b, s = x.shape
    nchunk = s // _LANE
    mask = (x != _PAD).astype(jnp.float32)
    m = mask.reshape(b * nchunk, _LANE)
    r = lax.broadcasted_iota(jnp.int32, (_LANE, _LANE), 0)
    c = lax.broadcasted_iota(jnp.int32, (_LANE, _LANE), 1)
    incl = (r <= c).astype(jnp.float32)
    y = jnp.dot(m, incl, preferred_element_type=jnp.float32)
    sums = jnp.sum(mask.reshape(b, nchunk, _LANE), axis=-1)
    r2 = lax.broadcasted_iota(jnp.int32, (nchunk, nchunk), 0)
    c2 = lax.broadcasted_iota(jnp.int32, (nchunk, nchunk), 1)
    excl = (r2 < c2).astype(jnp.float32)
    off = jnp.dot(sums, excl, preferred_element_type=jnp.float32)
    pos = y.reshape(b, nchunk, _LANE) + off[:, :, None]
    pos = pos.reshape(b, s) * mask + float(_PAD)
    o_ref[...] = pos.astype(jnp.int32)


def _gather_rows(table, idx):
    v, d = table.shape
    n = idx.shape[0]
    b_per_w = n // _NW
    mesh = plsc.VectorSubcoreMesh(core_axis_name="c", subcore_axis_name="s")

    @functools.partial(
        pl.kernel,
        mesh=mesh,
        out_type=jax.ShapeDtypeStruct((n, d), jnp.float32),
        scratch_types=[
            pltpu.VMEM((b_per_w,), jnp.int32),
            pltpu.VMEM((_CHUNK, d), jnp.float32),
            pltpu.VMEM((_CHUNK, d), jnp.float32),
            pltpu.SemaphoreType.DMA,
            pltpu.SemaphoreType.DMA,
            pltpu.SemaphoreType.DMA,
            pltpu.SemaphoreType.DMA,
        ],
    )
    def k(table_hbm, idx_hbm, out_hbm, idx_v, rows0, rows1, g0, g1, w0, w1):
        wid = lax.axis_index("s") * _NC + lax.axis_index("c")
        base = wid * b_per_w
        pltpu.sync_copy(idx_hbm.at[pl.ds(base, b_per_w)], idx_v)

        rows = (rows0, rows1)
        gsems = (g0, g1)
        wsems = (w0, w1)
        nchunks = b_per_w // _CHUNK

        def gather(i):
            return pltpu.async_copy(
                table_hbm.at[idx_v.at[pl.ds(i * _CHUNK, _CHUNK)]],
                rows[i % 2],
                gsems[i % 2],
            )

        def write(i):
            return pltpu.async_copy(
                rows[i % 2],
                out_hbm.at[pl.ds(base + i * _CHUNK, _CHUNK)],
                wsems[i % 2],
            )

        gh = [gather(0), gather(1)]
        wh = [None, None]
        for i in range(nchunks):
            b = i % 2
            gh[b].wait()
            wh[b] = write(i)
            if i + 2 < nchunks:
                wh[b].wait()
                gh[b] = gather(i + 2)
        wh[(nchunks - 2) % 2].wait()
        wh[(nchunks - 1) % 2].wait()

    return k(table, idx)


def kernel(input, table):
    b, s = input.shape
    positions = pl.pallas_call(
        _positions_body,
        out_shape=jax.ShapeDtypeStruct((b, s), jnp.int32),
    )(input)
    idx = positions.reshape(-1)
    out = _gather_rows(table, idx)
    return out.reshape(b, s, table.shape[1])
